# baseline (speedup 1.0000x reference)
import functools

import jax
import jax.numpy as jnp
from jax import lax
from jax.experimental import pallas as pl
from jax.experimental.pallas import tpu as pltpu
from jax.experimental.pallas import tpu_sc as plsc

N_SUBJECTS = 16
NB_BLK = 64


def _make_sc_expand(tot):
    info = plsc.get_sparse_core_info()
    nc, ns, nl = 1, info.num_subcores, info.num_lanes
    nw = nc * ns
    chunk = tot // nw
    mesh = plsc.VectorSubcoreMesh(core_axis_name="c", subcore_axis_name="s", num_cores=1)

    @functools.partial(
        pl.kernel,
        mesh=mesh,
        out_type=[
            jax.ShapeDtypeStruct((tot,), jnp.float32),
            jax.ShapeDtypeStruct((tot,), jnp.float32),
        ],
        scratch_types=[
            pltpu.VMEM((chunk,), jnp.int32),
            pltpu.VMEM((N_SUBJECTS,), jnp.float32),
            pltpu.VMEM((N_SUBJECTS,), jnp.float32),
            pltpu.VMEM((chunk,), jnp.float32),
            pltpu.VMEM((chunk,), jnp.float32),
        ],
    )
    def sc_expand(seg_hbm, tp0_hbm, tp1_hbm, te0_hbm, te1_hbm,
                  idx_v, t0_v, t1_v, o0_v, o1_v):
        wid = lax.axis_index("s") * nc + lax.axis_index("c")
        base = wid * chunk
        pltpu.sync_copy(seg_hbm.at[pl.ds(base, chunk)], idx_v)
        pltpu.sync_copy(tp0_hbm, t0_v)
        pltpu.sync_copy(tp1_hbm, t1_v)
        tbl0 = t0_v[...]
        tbl1 = t1_v[...]

        for i in range(chunk // nl):
            sl = pl.ds(i * nl, nl)
            idx = idx_v[sl]
            o0_v[sl] = tbl0.at[idx].get(mode="promise_in_bounds")
            o1_v[sl] = tbl1.at[idx].get(mode="promise_in_bounds")
        pltpu.sync_copy(o0_v, te0_hbm.at[pl.ds(base, chunk)])
        pltpu.sync_copy(o1_v, te1_hbm.at[pl.ds(base, chunk)])

    return sc_expand


def _fma_body(te0_ref, te1_ref, x_ref, o_ref):
    y = x_ref[...] * te1_ref[0] + te0_ref[0]
    o_ref[...] = y.reshape(o_ref.shape)


def kernel(x, segment_ids, time_parameters0, time_parameters1):
    nb, tot = x.shape
    n_blocks = nb // NB_BLK
    rows_per_blk = NB_BLK * tot // 1024
    seg = segment_ids.astype(jnp.int32)
    tp0 = time_parameters0.reshape(N_SUBJECTS)
    tp1 = time_parameters1.reshape(N_SUBJECTS)

    te0, te1 = _make_sc_expand(tot)(seg, tp0, tp1)
    te0_3 = te0.reshape(1, 1, tot)
    te1_3 = te1.reshape(1, 1, tot)

    out = pl.pallas_call(
        _fma_body,
        grid=(n_blocks,),
        in_specs=[
            pl.BlockSpec((1, 1, tot), lambda i: (0, 0, 0)),
            pl.BlockSpec((1, 1, tot), lambda i: (0, 0, 0)),
            pl.BlockSpec((NB_BLK, tot), lambda i: (i, 0)),
        ],
        out_specs=pl.BlockSpec((rows_per_blk, 8, 128), lambda i: (i, 0, 0)),
        out_shape=jax.ShapeDtypeStruct((nb * tot // 1024, 8, 128), jnp.float32),
    )(te0_3, te1_3, x)
    return out.reshape(nb, tot, 1)
